# baseline (speedup 1.0000x reference)
import functools

import jax
import jax.numpy as jnp
from jax import lax
from jax.experimental import pallas as pl
from jax.experimental.pallas import tpu as pltpu
from jax.experimental.pallas import tpu_sc as plsc

N_NODES = 10000
N_EDGES = 320000
D = 128

NUM_CORES = 2
NUM_SUBCORES = 16
NUM_WORKERS = NUM_CORES * NUM_SUBCORES
CHUNK = 128
CHUNKS = 78
LEFT_BASE = NUM_WORKERS * CHUNKS
U = 6
OUTER = 72 // U
NROWS = 2
IDXR = 6
ROWS_PER_TILE = 624
TAIL_ROWS = N_NODES - NUM_SUBCORES * ROWS_PER_TILE

ROW_BLOCK = 2000
NBLK = N_NODES // ROW_BLOCK


def _finish_body(a_ref, b2_ref, x_ref, c0_ref, c1_ref, w_ref, b_ref, o_ref):
    agg = a_ref[0] + b2_ref[0] - x_ref[...]
    cnt = c0_ref[0, 0] + c1_ref[0, 0] + 1.0
    o_ref[...] = (
        lax.dot_general(
            agg, w_ref[...],
            (((1,), (1,)), ((), ())),
            preferred_element_type=jnp.float32,
        )
        + cnt * b_ref[...]
    )


def _finish(partials, cnt, x, W, b):
    c3 = cnt.reshape(NUM_CORES, NBLK, ROW_BLOCK, 1)
    return pl.pallas_call(
        _finish_body,
        grid=(NBLK,),
        in_specs=[
            pl.BlockSpec((1, ROW_BLOCK, D), lambda i: (0, i, 0)),
            pl.BlockSpec((1, ROW_BLOCK, D), lambda i: (1, i, 0)),
            pl.BlockSpec((ROW_BLOCK, D), lambda i: (i, 0)),
            pl.BlockSpec((1, 1, ROW_BLOCK, 1), lambda i: (0, i, 0, 0)),
            pl.BlockSpec((1, 1, ROW_BLOCK, 1), lambda i: (1, i, 0, 0)),
            pl.BlockSpec((D, D), lambda i: (0, 0)),
            pl.BlockSpec((1, D), lambda i: (0, 0)),
        ],
        out_specs=pl.BlockSpec((ROW_BLOCK, D), lambda i: (i, 0)),
        out_shape=jax.ShapeDtypeStruct((N_NODES, D), jnp.float32),
    )(partials, partials, x, c3, c3, W, b.reshape(1, D))


def _scatter_body(x_hbm, ei_hbm, out_hbm, cnt_hbm,
                  idx, rows, ones, zbuf, acc, cacc,
                  isem, gsem, ssem, osem):
    cid = lax.axis_index("c")
    sid = lax.axis_index("s")
    wid = sid * NUM_CORES + cid
    row_base = sid * ROWS_PER_TILE
    cbase = wid * CHUNKS

    def fire_idx(c, slot):
        pltpu.async_copy(
            ei_hbm.at[:, pl.ds((cbase + c) * CHUNK, CHUNK)],
            idx.at[slot], isem)

    def wait_idx():
        pltpu.make_async_copy(
            ei_hbm.at[:, pl.ds(0, CHUNK)], idx.at[0], isem).wait()

    def wait_gather(b):
        pltpu.make_async_copy(x_hbm.at[idx.at[0, 0]], rows[b], gsem).wait()

    def wait_scatter(b):
        pltpu.make_async_copy(rows[b], acc.at[idx.at[0, 1]], ssem).wait()

    def wait_ones():
        pltpu.make_async_copy(ones, cacc.at[idx.at[0, 1]], osem).wait()

    def fire_chunk(b, slot):
        pltpu.async_copy(rows[b], acc.at[idx.at[slot, 1]], ssem, add=True)
        pltpu.async_copy(ones, cacc.at[idx.at[slot, 1]], osem, add=True)

    for c in range(4):
        fire_idx(c, c)

    def setvec(i, _):
        ones[pl.ds(i * 16, 16)] = jnp.ones((16,), jnp.float32)
        zbuf[pl.ds(i * 16, 16)] = jnp.zeros((16,), jnp.float32)
        return 0
    lax.fori_loop(0, CHUNK // 16, setvec, 0)

    pltpu.sync_copy(x_hbm.at[pl.ds(row_base, ROWS_PER_TILE)],
                    acc.at[pl.ds(row_base, ROWS_PER_TILE)])
    for z in range(ROWS_PER_TILE // CHUNK):
        pltpu.sync_copy(zbuf, cacc.at[pl.ds(row_base + z * CHUNK, CHUNK)])
    pltpu.sync_copy(zbuf.at[pl.ds(0, ROWS_PER_TILE % CHUNK)],
                    cacc.at[pl.ds(row_base + 4 * CHUNK,
                                  ROWS_PER_TILE % CHUNK)])

    @pl.when(sid == NUM_SUBCORES - 1)
    def _():
        pltpu.sync_copy(
            x_hbm.at[pl.ds(NUM_SUBCORES * ROWS_PER_TILE, TAIL_ROWS)],
            acc.at[pl.ds(NUM_SUBCORES * ROWS_PER_TILE, TAIL_ROWS)])
        pltpu.sync_copy(
            zbuf.at[pl.ds(0, TAIL_ROWS)],
            cacc.at[pl.ds(NUM_SUBCORES * ROWS_PER_TILE, TAIL_ROWS)])

    wait_idx()
    pltpu.async_copy(x_hbm.at[idx.at[0, 0]], rows[0], gsem)

    plsc.subcore_barrier()

    def step(j, u, guard_lo, fire_i, fire_g):
        if guard_lo:
            @pl.when(j >= 1)
            def _():
                wait_scatter((u + 1) % NROWS)
        else:
            wait_scatter((u + 1) % NROWS)

        if fire_i:
            fire_idx(j + 4, (u + 4) % IDXR)

        if fire_g:
            wait_idx()
            pltpu.async_copy(x_hbm.at[idx.at[(u + 1) % IDXR, 0]],
                             rows[(u + 1) % NROWS], gsem)
        wait_gather(u % NROWS)
        fire_chunk(u % NROWS, u % IDXR)

    def outer(jo, _):
        for u in range(U):
            step(jo * U + u, u, guard_lo=True, fire_i=True, fire_g=True)
        return 0

    lax.fori_loop(0, OUTER, outer, 0)

    step(72, 0, guard_lo=False, fire_i=True, fire_g=True)
    step(73, 1, guard_lo=False, fire_i=True, fire_g=True)
    step(74, 2, guard_lo=False, fire_i=False, fire_g=True)
    step(75, 3, guard_lo=False, fire_i=False, fire_g=True)
    step(76, 4, guard_lo=False, fire_i=False, fire_g=True)
    step(77, 5, guard_lo=False, fire_i=False, fire_g=False)

    wait_scatter(0)
    for _ in range(CHUNKS):
        wait_ones()

    @pl.when(wid < 2500 - LEFT_BASE)
    def _():
        pltpu.sync_copy(
            ei_hbm.at[:, pl.ds((LEFT_BASE + wid) * CHUNK, CHUNK)],
            idx.at[0])
        pltpu.async_copy(x_hbm.at[idx.at[0, 0]], rows[0], gsem).wait()
        fire_chunk(0, 0)
        wait_scatter(0)
        wait_ones()


    plsc.subcore_barrier()

    pltpu.sync_copy(acc.at[pl.ds(row_base, ROWS_PER_TILE)],
                    out_hbm.at[cid, pl.ds(row_base, ROWS_PER_TILE)])
    for z in range(ROWS_PER_TILE // CHUNK):
        pltpu.sync_copy(cacc.at[pl.ds(row_base + z * CHUNK, CHUNK)], zbuf)
        pltpu.sync_copy(zbuf, cnt_hbm.at[pl.ds(cid * N_NODES + row_base
                                               + z * CHUNK, CHUNK)])
    pltpu.sync_copy(cacc.at[pl.ds(row_base + 4 * CHUNK,
                                  ROWS_PER_TILE % CHUNK)],
                    zbuf.at[pl.ds(0, ROWS_PER_TILE % CHUNK)])
    pltpu.sync_copy(zbuf.at[pl.ds(0, ROWS_PER_TILE % CHUNK)],
                    cnt_hbm.at[pl.ds(cid * N_NODES + row_base + 4 * CHUNK,
                                     ROWS_PER_TILE % CHUNK)])

    @pl.when(sid == NUM_SUBCORES - 1)
    def _():
        pltpu.sync_copy(
            acc.at[pl.ds(NUM_SUBCORES * ROWS_PER_TILE, TAIL_ROWS)],
            out_hbm.at[cid, pl.ds(NUM_SUBCORES * ROWS_PER_TILE, TAIL_ROWS)])
        pltpu.sync_copy(
            cacc.at[pl.ds(NUM_SUBCORES * ROWS_PER_TILE, TAIL_ROWS)],
            zbuf.at[pl.ds(0, TAIL_ROWS)])
        pltpu.sync_copy(
            zbuf.at[pl.ds(0, TAIL_ROWS)],
            cnt_hbm.at[pl.ds(cid * N_NODES + NUM_SUBCORES * ROWS_PER_TILE,
                             TAIL_ROWS)])


@functools.partial(
    pl.kernel,
    out_type=(jax.ShapeDtypeStruct((NUM_CORES, N_NODES, D), jnp.float32),
              jax.ShapeDtypeStruct((NUM_CORES * N_NODES,), jnp.float32)),
    mesh=plsc.VectorSubcoreMesh(
        core_axis_name="c", subcore_axis_name="s",
        num_cores=NUM_CORES, num_subcores=NUM_SUBCORES),
    scratch_types=[
        pltpu.VMEM((IDXR, 2, CHUNK), jnp.int32),
        [pltpu.VMEM((CHUNK, D), jnp.float32) for _ in range(NROWS)],
        pltpu.VMEM((CHUNK,), jnp.float32),
        pltpu.VMEM((CHUNK,), jnp.float32),
        pltpu.VMEM_SHARED((N_NODES, D), jnp.float32),
        pltpu.VMEM_SHARED((N_NODES,), jnp.float32),
        pltpu.SemaphoreType.DMA,
        pltpu.SemaphoreType.DMA,
        pltpu.SemaphoreType.DMA,
        pltpu.SemaphoreType.DMA,
    ],
)
def _scatter_add(x_hbm, ei_hbm, out_hbm, cnt_hbm,
                 idx, rows, ones, zbuf, acc, cacc,
                 isem, gsem, ssem, osem):
    _scatter_body(x_hbm, ei_hbm, out_hbm, cnt_hbm,
                  idx, rows, ones, zbuf, acc, cacc,
                  isem, gsem, ssem, osem)


def kernel(x, edge_index, W, b):
    partials, cnt = _scatter_add(x, edge_index.astype(jnp.int32))
    return _finish(partials, cnt, x, W, b)


# speedup vs baseline: 1.1996x; 1.1996x over previous
import functools

import jax
import jax.numpy as jnp
from jax import lax
from jax.experimental import pallas as pl
from jax.experimental.pallas import tpu as pltpu
from jax.experimental.pallas import tpu_sc as plsc

N_NODES = 10000
N_EDGES = 320000
D = 128

NUM_CORES = 2
NUM_SUBCORES = 16
NUM_WORKERS = NUM_CORES * NUM_SUBCORES
CHUNK = 128
CHUNKS = 78
LEFT_BASE = NUM_WORKERS * CHUNKS
U = 6
OUTER = 72 // U
NROWS = 3
IDXR = 6
ROWS_PER_TILE = 624
TAIL_ROWS = N_NODES - NUM_SUBCORES * ROWS_PER_TILE

ROW_BLOCK = 2000


def _matmul_body(x_ref, w_ref, b_ref, h_ref):
    h_ref[...] = (
        lax.dot_general(
            x_ref[...], w_ref[...],
            (((1,), (1,)), ((), ())),
            preferred_element_type=jnp.float32,
        )
        + b_ref[...]
    )


def _linear(x, W, b):
    grid = N_NODES // ROW_BLOCK
    return pl.pallas_call(
        _matmul_body,
        grid=(grid,),
        in_specs=[
            pl.BlockSpec((ROW_BLOCK, D), lambda i: (i, 0)),
            pl.BlockSpec((D, D), lambda i: (0, 0)),
            pl.BlockSpec((1, D), lambda i: (0, 0)),
        ],
        out_specs=pl.BlockSpec((ROW_BLOCK, D), lambda i: (i, 0)),
        out_shape=jax.ShapeDtypeStruct((N_NODES, D), jnp.float32),
    )(x, W, b.reshape(1, D))


def _combine_body(a_ref, b_ref, h_ref, o_ref):
    o_ref[...] = a_ref[0] + b_ref[0] - h_ref[...]


def _combine(partials, h):
    grid = N_NODES // ROW_BLOCK
    return pl.pallas_call(
        _combine_body,
        grid=(grid,),
        in_specs=[
            pl.BlockSpec((1, ROW_BLOCK, D), lambda i: (0, i, 0)),
            pl.BlockSpec((1, ROW_BLOCK, D), lambda i: (1, i, 0)),
            pl.BlockSpec((ROW_BLOCK, D), lambda i: (i, 0)),
        ],
        out_specs=pl.BlockSpec((ROW_BLOCK, D), lambda i: (i, 0)),
        out_shape=jax.ShapeDtypeStruct((N_NODES, D), jnp.float32),
    )(partials, partials, h)


def _scatter_body(h_hbm, ei_hbm, out_hbm, idx, rows, acc, isem, gsem, ssem):
    cid = lax.axis_index("c")
    sid = lax.axis_index("s")
    wid = sid * NUM_CORES + cid
    row_base = sid * ROWS_PER_TILE
    cbase = wid * CHUNKS

    def fire_idx(c, slot):
        pltpu.async_copy(
            ei_hbm.at[:, pl.ds((cbase + c) * CHUNK, CHUNK)],
            idx.at[slot], isem)

    def wait_idx():
        pltpu.make_async_copy(
            ei_hbm.at[:, pl.ds(0, CHUNK)], idx.at[0], isem).wait()

    def wait_gather(b):
        pltpu.make_async_copy(h_hbm.at[idx.at[0, 0]], rows[b], gsem).wait()

    def wait_scatter(b):
        pltpu.make_async_copy(rows[b], acc.at[idx.at[0, 1]], ssem).wait()

    for c in range(4):
        fire_idx(c, c)

    pltpu.sync_copy(h_hbm.at[pl.ds(row_base, ROWS_PER_TILE)],
                    acc.at[pl.ds(row_base, ROWS_PER_TILE)])

    @pl.when(sid == NUM_SUBCORES - 1)
    def _():
        pltpu.sync_copy(
            h_hbm.at[pl.ds(NUM_SUBCORES * ROWS_PER_TILE, TAIL_ROWS)],
            acc.at[pl.ds(NUM_SUBCORES * ROWS_PER_TILE, TAIL_ROWS)])

    wait_idx()
    pltpu.async_copy(h_hbm.at[idx.at[0, 0]], rows[0], gsem)

    plsc.subcore_barrier()

    def step(j, u, guard_lo, fire_i, fire_g):
        if guard_lo:
            @pl.when(j >= 2)
            def _():
                wait_scatter((u + 1) % NROWS)
        else:
            wait_scatter((u + 1) % NROWS)

        if fire_i:
            fire_idx(j + 4, (u + 4) % IDXR)

        if fire_g:
            wait_idx()
            pltpu.async_copy(h_hbm.at[idx.at[(u + 1) % IDXR, 0]],
                             rows[(u + 1) % NROWS], gsem)
        wait_gather(u % NROWS)
        pltpu.async_copy(rows[u % NROWS], acc.at[idx.at[u % IDXR, 1]],
                         ssem, add=True)

    def outer(jo, _):
        for u in range(U):
            step(jo * U + u, u, guard_lo=True, fire_i=True, fire_g=True)
        return 0

    lax.fori_loop(0, OUTER, outer, 0)

    step(72, 0, guard_lo=False, fire_i=True, fire_g=True)
    step(73, 1, guard_lo=False, fire_i=True, fire_g=True)
    step(74, 2, guard_lo=False, fire_i=False, fire_g=True)
    step(75, 3, guard_lo=False, fire_i=False, fire_g=True)
    step(76, 4, guard_lo=False, fire_i=False, fire_g=True)
    step(77, 5, guard_lo=False, fire_i=False, fire_g=False)

    for b in range(2):
        wait_scatter(b)

    @pl.when(wid < 2500 - LEFT_BASE)
    def _():
        pltpu.sync_copy(
            ei_hbm.at[:, pl.ds((LEFT_BASE + wid) * CHUNK, CHUNK)],
            idx.at[0])
        pltpu.async_copy(h_hbm.at[idx.at[0, 0]], rows[0], gsem).wait()
        pltpu.async_copy(rows[0], acc.at[idx.at[0, 1]], ssem, add=True)
        wait_scatter(0)

    plsc.subcore_barrier()

    pltpu.sync_copy(acc.at[pl.ds(row_base, ROWS_PER_TILE)],
                    out_hbm.at[cid, pl.ds(row_base, ROWS_PER_TILE)])

    @pl.when(sid == NUM_SUBCORES - 1)
    def _():
        pltpu.sync_copy(
            acc.at[pl.ds(NUM_SUBCORES * ROWS_PER_TILE, TAIL_ROWS)],
            out_hbm.at[cid, pl.ds(NUM_SUBCORES * ROWS_PER_TILE, TAIL_ROWS)])


@functools.partial(
    pl.kernel,
    out_type=jax.ShapeDtypeStruct((NUM_CORES, N_NODES, D), jnp.float32),
    mesh=plsc.VectorSubcoreMesh(
        core_axis_name="c", subcore_axis_name="s",
        num_cores=NUM_CORES, num_subcores=NUM_SUBCORES),
    scratch_types=[
        pltpu.VMEM((IDXR, 2, CHUNK), jnp.int32),
        [pltpu.VMEM((CHUNK, D), jnp.float32) for _ in range(NROWS)],
        pltpu.VMEM_SHARED((N_NODES, D), jnp.float32),
        pltpu.SemaphoreType.DMA,
        pltpu.SemaphoreType.DMA,
        pltpu.SemaphoreType.DMA,
    ],
)
def _scatter_add(h_hbm, ei_hbm, out_hbm, idx, rows, acc, isem, gsem, ssem):
    _scatter_body(h_hbm, ei_hbm, out_hbm, idx, rows, acc, isem, gsem, ssem)


def kernel(x, edge_index, W, b):
    h = _linear(x, W, b)
    partials = _scatter_add(h, edge_index.astype(jnp.int32))
    return _combine(partials, h)
